# Initial kernel scaffold; baseline (speedup 1.0000x reference)
import jax
import jax.numpy as jnp
from jax.experimental import pallas as pl


def kernel(inputs, table):
    raise NotImplementedError("write your pallas kernel here")


# speedup vs baseline: 7.1172x; 7.1172x over previous
import jax
import jax.numpy as jnp
from jax.experimental import pallas as pl
from jax.experimental.pallas import tpu as pltpu

BATCH = 16384
SEQ = 200
EMBED_DIM = 128
PAD_IDX = 2

TOTAL = BATCH * SEQ
BLOCK_ROWS = 8192
GRID = TOTAL // BLOCK_ROWS


def _embed_body(idx_ref, table_ref, out_ref):
    idx = idx_ref[...]
    t0 = table_ref[0:1, :]
    t1 = table_ref[1:2, :]
    e0 = (idx == 0).astype(jnp.float32)
    e1 = (idx == 1).astype(jnp.float32)
    out_ref[...] = e0 * t0 + e1 * t1


def kernel(inputs, table):
    idx = inputs.reshape(TOTAL, 1)
    out = pl.pallas_call(
        _embed_body,
        grid=(GRID,),
        in_specs=[
            pl.BlockSpec((BLOCK_ROWS, 1), lambda i: (i, 0)),
            pl.BlockSpec((3, EMBED_DIM), lambda i: (0, 0)),
        ],
        out_specs=pl.BlockSpec((BLOCK_ROWS, EMBED_DIM), lambda i: (i, 0)),
        out_shape=jax.ShapeDtypeStruct((TOTAL, EMBED_DIM), jnp.float32),
        compiler_params=pltpu.CompilerParams(
            dimension_semantics=("arbitrary",),
        ),
    )(idx, table)
    return out.reshape(BATCH, SEQ, EMBED_DIM)
